# baseline (speedup 1.0000x reference)
import functools

import jax
import jax.numpy as jnp
from jax import lax
from jax.experimental import pallas as pl
from jax.experimental.pallas import tpu as pltpu
from jax.experimental.pallas import tpu_sc as plsc

_VOCAB = 50000
_TOTAL = 50512
_N = 4096
_EPS = 1e-20
_IGNORE = -100

_NC, _NS = 2, 16
_NW = _NC * _NS
_C = _N // _NW
_L = 16

_mesh = plsc.VectorSubcoreMesh(core_axis_name="c", subcore_axis_name="s")


@functools.partial(
    pl.kernel,
    mesh=_mesh,
    out_type=jax.ShapeDtypeStruct((_N,), jnp.float32),
    scratch_types=[
        pltpu.VMEM((_C,), jnp.int32),
        pltpu.VMEM((_C,), jnp.int32),
        pltpu.VMEM((_C,), jnp.int32),
        pltpu.VMEM((_C,), jnp.int32),
        pltpu.VMEM((_C,), jnp.float32),
        pltpu.VMEM((_C,), jnp.float32),
        pltpu.VMEM((_C,), jnp.float32),
        pltpu.SemaphoreType.DMA,
    ],
)
def _gather_probs(scores_hbm, align_hbm, target_hbm, out_hbm,
                  tgt_v, aln_v, ti_v, ci_v, vp_v, cp_v, o_v, sem):
    wid = lax.axis_index("s") * _NC + lax.axis_index("c")
    base = wid * _C
    pltpu.sync_copy(target_hbm.at[pl.ds(base, _C)], tgt_v)
    pltpu.sync_copy(align_hbm.at[pl.ds(base, _C)], aln_v)
    lane = lax.iota(jnp.int32, _L) * _TOTAL
    for j in range(_C // _L):
        sl = pl.ds(j * _L, _L)
        rowb = (base + j * _L) * _TOTAL + lane
        ti_v[sl] = rowb + tgt_v[sl]
        ci_v[sl] = rowb + (_VOCAB + aln_v[sl])
    g1 = pltpu.async_copy(scores_hbm.at[ti_v], vp_v, sem)
    g2 = pltpu.async_copy(scores_hbm.at[ci_v], cp_v, sem)
    g1.wait()
    g2.wait()
    for j in range(_C // _L):
        sl = pl.ds(j * _L, _L)
        a = aln_v[sl]
        t = tgt_v[sl]
        c = jnp.where(a == 0, 0.0, cp_v[sl]) + _EPS
        non_copy = (a == 0) | (t != 0)
        o_v[sl] = jnp.where(non_copy, c + vp_v[sl], c)
    pltpu.sync_copy(o_v, out_hbm.at[pl.ds(base, _C)])


def _loss_body(p_ref, t_ref, o_ref):
    loss = -jnp.log(p_ref[...])
    o_ref[...] = jnp.where(t_ref[...] == _IGNORE, 0.0, loss)


def kernel(scores, align, target):
    probs = _gather_probs(scores.reshape(-1), align, target)
    loss = pl.pallas_call(
        _loss_body,
        out_shape=jax.ShapeDtypeStruct((_N // 128, 128), jnp.float32),
    )(probs.reshape(_N // 128, 128), target.reshape(_N // 128, 128))
    return loss.reshape(_N)


# speedup vs baseline: 2.2635x; 2.2635x over previous
import functools

import jax
import jax.numpy as jnp
from jax import lax
from jax.experimental import pallas as pl
from jax.experimental.pallas import tpu as pltpu
from jax.experimental.pallas import tpu_sc as plsc

_VOCAB = 50000
_TOTAL = 50512
_N = 4096
_EPS = 1e-20
_IGNORE = -100

_NC, _NS = 2, 16
_NW = _NC * _NS
_C = _N // _NW
_L = 16
_P = 32
_W0 = 49920
_WW = _TOTAL - _W0

_mesh = plsc.VectorSubcoreMesh(core_axis_name="c", subcore_axis_name="s")


@functools.partial(
    pl.kernel,
    mesh=_mesh,
    compiler_params=pltpu.CompilerParams(needs_layout_passes=False),
    out_type=jax.ShapeDtypeStruct((_N,), jnp.float32),
    scratch_types=[
        pltpu.VMEM((_C,), jnp.int32),
        pltpu.VMEM((_C,), jnp.int32),
        pltpu.VMEM((_P, 8, 128), jnp.float32),
        pltpu.VMEM((_C, _WW), jnp.float32),
        pltpu.VMEM((_C,), jnp.float32),
        pltpu.VMEM((_C,), jnp.float32),
        pltpu.SemaphoreType.DMA,
        pltpu.SemaphoreType.DMA,
    ],
)
def _gather_probs(scores_hbm, align_hbm, target_hbm, out_hbm,
                  tgt_v, aln_v, ch_v, ch_c, vp_v, o_v, sem, sem2):
    wid = lax.axis_index("s") * _NC + lax.axis_index("c")
    base = pl.multiple_of(wid * _C, _C)
    pltpu.sync_copy(target_hbm.at[pl.ds(base, _C)], tgt_v)
    pltpu.sync_copy(align_hbm.at[pl.ds(base, _C)], aln_v)

    cw = pltpu.async_copy(
        scores_hbm.at[pl.ds(base, _C), pl.ds(_W0, _WW)], ch_c, sem2)

    for p in range(_C // _P):
        copies = []
        for j in range(_P // _L):
            cvec = (tgt_v[pl.ds(p * _P + j * _L, _L)] >> 7) << 7
            for i in range(_L):
                r = j * _L + i
                rr = p * _P + r
                row0 = pl.multiple_of(base + (rr & ~7), 8)
                c0 = pl.multiple_of(cvec[i], 128)
                copies.append(pltpu.async_copy(
                    scores_hbm.at[pl.ds(row0, 8), pl.ds(c0, 128)],
                    ch_v.at[r], sem))
        for cp in copies:
            cp.wait()
        for j in range(_P // _L):
            rl = lax.iota(jnp.int32, _L) + j * _L
            sl = pl.ds(p * _P + j * _L, _L)
            t = tgt_v[sl]
            rr = rl + p * _P
            vp_v[sl] = plsc.load_gather(ch_v, [rl, rr & 7, t & 127])

    cw.wait()
    for j in range(_C // _L):
        sl = pl.ds(j * _L, _L)
        r = lax.iota(jnp.int32, _L) + j * _L
        t = tgt_v[sl]
        a = aln_v[sl]
        c = plsc.load_gather(ch_c, [r, a + (_VOCAB - _W0)])
        c = jnp.where(a == 0, 0.0, c) + _EPS
        non_copy = (a == 0) | (t != 0)
        o_v[sl] = jnp.where(non_copy, c + vp_v[sl], c)
    pltpu.sync_copy(o_v, out_hbm.at[pl.ds(base, _C)])


def _loss_body(p_ref, t_ref, o_ref):
    loss = -jnp.log(p_ref[...])
    o_ref[...] = jnp.where(t_ref[...] == _IGNORE, 0.0, loss)


def kernel(scores, align, target):
    probs = _gather_probs(scores, align, target)
    loss = pl.pallas_call(
        _loss_body,
        out_shape=jax.ShapeDtypeStruct((_N // 128, 128), jnp.float32),
    )(probs.reshape(_N // 128, 128), target.reshape(_N // 128, 128))
    return loss.reshape(_N)


# speedup vs baseline: 47.6314x; 21.0434x over previous
import functools

import jax
import jax.numpy as jnp
from jax import lax
from jax.experimental import pallas as pl
from jax.experimental.pallas import tpu as pltpu
from jax.experimental.pallas import tpu_sc as plsc

_VOCAB = 50000
_TOTAL = 50512
_N = 4096
_EPS = 1e-20
_IGNORE = -100

_NC, _NS = 2, 16
_NW = _NC * _NS
_C = _N // _NW
_L = 16
_P = 32
_W0 = 49920
_WW = _TOTAL - _W0

_mesh = plsc.VectorSubcoreMesh(core_axis_name="c", subcore_axis_name="s")


@functools.partial(
    pl.kernel,
    mesh=_mesh,
    compiler_params=pltpu.CompilerParams(needs_layout_passes=False),
    out_type=jax.ShapeDtypeStruct((_N,), jnp.float32),
    scratch_types=[
        pltpu.VMEM((_C,), jnp.int32),
        pltpu.VMEM((_C,), jnp.int32),
        pltpu.VMEM((_P, 8, _C), jnp.float32),
        pltpu.VMEM((_WW, _C), jnp.float32),
        pltpu.VMEM((_C,), jnp.float32),
        pltpu.VMEM((_C,), jnp.float32),
        pltpu.SemaphoreType.DMA,
        pltpu.SemaphoreType.DMA,
    ],
)
def _gather_probs(scores_t_hbm, align_hbm, target_hbm, out_hbm,
                  tgt_v, aln_v, ch_v, ch_c, vp_v, o_v, sem, sem2):
    wid = lax.axis_index("s") * _NC + lax.axis_index("c")
    base = pl.multiple_of(wid * _C, _C)
    pltpu.sync_copy(target_hbm.at[pl.ds(base, _C)], tgt_v)
    pltpu.sync_copy(align_hbm.at[pl.ds(base, _C)], aln_v)

    cw = pltpu.async_copy(
        scores_t_hbm.at[pl.ds(_W0, _WW), pl.ds(base, _C)], ch_c, sem2)

    for p in range(_C // _P):
        copies = []
        for j in range(_P // _L):
            tvec = (tgt_v[pl.ds(p * _P + j * _L, _L)] >> 3) << 3
            for i in range(_L):
                r = j * _L + i
                t0 = pl.multiple_of(tvec[i], 8)
                copies.append(pltpu.async_copy(
                    scores_t_hbm.at[pl.ds(t0, 8), pl.ds(base, _C)],
                    ch_v.at[r], sem))
        for cp in copies:
            cp.wait()
        for j in range(_P // _L):
            rl = lax.iota(jnp.int32, _L) + j * _L
            sl = pl.ds(p * _P + j * _L, _L)
            t = tgt_v[sl]
            rr = rl + p * _P
            vp_v[sl] = plsc.load_gather(ch_v, [rl, t & 7, rr])

    cw.wait()
    for j in range(_C // _L):
        sl = pl.ds(j * _L, _L)
        r = lax.iota(jnp.int32, _L) + j * _L
        t = tgt_v[sl]
        a = aln_v[sl]
        c = plsc.load_gather(ch_c, [a + (_VOCAB - _W0), r])
        c = jnp.where(a == 0, 0.0, c) + _EPS
        non_copy = (a == 0) | (t != 0)
        o_v[sl] = jnp.where(non_copy, c + vp_v[sl], c)
    pltpu.sync_copy(o_v, out_hbm.at[pl.ds(base, _C)])


def _loss_body(p_ref, t_ref, o_ref):
    loss = -jnp.log(p_ref[...])
    o_ref[...] = jnp.where(t_ref[...] == _IGNORE, 0.0, loss)


def kernel(scores, align, target):
    probs = _gather_probs(scores.T, align, target)
    loss = pl.pallas_call(
        _loss_body,
        out_shape=jax.ShapeDtypeStruct((_N // 128, 128), jnp.float32),
    )(probs.reshape(_N // 128, 128), target.reshape(_N // 128, 128))
    return loss.reshape(_N)
